# baseline (speedup 1.0000x reference)
import functools

import jax
import jax.numpy as jnp
from jax import lax
from jax.experimental import pallas as pl
from jax.experimental.pallas import tpu as pltpu
from jax.experimental.pallas import tpu_sc as plsc

B = 16384
V = 100000
D = 16
NU = 13
NI = 13
H1, H2 = 256, 128

_NC, _NS = 2, 16
_NW = _NC * _NS
_BPW = B // _NW
_CH = 128
_NCH = _BPW // _CH
_RPC = _CH * NU
_JPC = _RPC // 128
_JPW = _NCH * _JPC
_BB = 2048


def _pool_body(ut, it, uidx, iidx, out, uidx_v, iidx_v, urows, irows,
               uacc, iacc, sem0, sem1):
    w = lax.axis_index("c") * _NS + lax.axis_index("s")
    pltpu.sync_copy(uidx.at[w], uidx_v)
    pltpu.sync_copy(iidx.at[w], iidx_v)
    sems = (sem0, sem1)

    def fire(c):
        buf = c % 2
        cps = []
        for j in range(_JPC):
            cps.append(pltpu.async_copy(
                ut.at[uidx_v.at[c * _JPC + j]],
                urows.at[buf, pl.ds(j * 128, 128), :], sems[buf]))
            cps.append(pltpu.async_copy(
                it.at[iidx_v.at[c * _JPC + j]],
                irows.at[buf, pl.ds(j * 128, 128), :], sems[buf]))
        return cps

    def accumulate(c):
        buf = c % 2

        def body(s, carry):
            base = s * NU
            u = urows[buf, base, :]
            v = irows[buf, base, :]
            for f in range(1, NU):
                u = u + urows[buf, base + f, :]
                v = v + irows[buf, base + f, :]
            uacc[s, :] = u
            iacc[s, :] = v
            return carry

        lax.fori_loop(0, _CH, body, 0, unroll=2)
        row0 = w * _BPW + c * _CH
        pltpu.sync_copy(uacc, out.at[0, pl.ds(row0, _CH), :])
        pltpu.sync_copy(iacc, out.at[1, pl.ds(row0, _CH), :])

    pending = fire(0)
    for c in range(_NCH):
        nxt = fire(c + 1) if c + 1 < _NCH else []
        for cp in pending:
            cp.wait()
        pending = nxt
        accumulate(c)


def _pool(ut, it, uidx, iidx):
    mesh = plsc.VectorSubcoreMesh(core_axis_name="c", subcore_axis_name="s")
    f = pl.kernel(
        _pool_body,
        out_type=jax.ShapeDtypeStruct((2, B, D), jnp.float32),
        mesh=mesh,
        scratch_types=[
            pltpu.VMEM((_JPW, 128), jnp.int32),
            pltpu.VMEM((_JPW, 128), jnp.int32),
            pltpu.VMEM((2, _RPC, D), jnp.float32),
            pltpu.VMEM((2, _RPC, D), jnp.float32),
            pltpu.VMEM((_CH, D), jnp.float32),
            pltpu.VMEM((_CH, D), jnp.float32),
            pltpu.SemaphoreType.DMA,
            pltpu.SemaphoreType.DMA,
        ],
        compiler_params=pltpu.CompilerParams(use_tc_tiling_on_sc=False),
    )
    return f(ut, it, uidx, iidx)


def _mlp_body(x, cW1, cb1, cW2, cb2, cW3, cb3, vW1, vb1, vW2, vb2, vW3, vb3,
              out):
    u = x[0]
    t = x[1]

    def tower(W1, b1, W2, b2, W3, b3):
        h = jnp.dot(u, W1[0:D, :], preferred_element_type=jnp.float32)
        h = h + jnp.dot(t, W1[D:2 * D, :], preferred_element_type=jnp.float32)
        h = jnp.maximum(h + b1[0, :], 0.0)
        h = jnp.dot(h, W2[...], preferred_element_type=jnp.float32) + b2[0, :]
        h = jnp.maximum(h, 0.0)
        o = jnp.dot(h, W3[...], preferred_element_type=jnp.float32)
        return o[:, 0] + b3[0, 0]

    ctr = jax.nn.sigmoid(tower(cW1, cb1, cW2, cb2, cW3, cb3))
    cvr = jax.nn.sigmoid(tower(vW1, vb1, vW2, vb2, vW3, vb3))
    out[0, :] = ctr
    out[1, :] = ctr * cvr


def _mlp(pooled, cW1, cb1, cW2, cb2, cW3, cb3, vW1, vb1, vW2, vb2, vW3, vb3):
    full = lambda shape: pl.BlockSpec(shape, lambda i: (0,) * len(shape))
    return pl.pallas_call(
        _mlp_body,
        grid=(B // _BB,),
        in_specs=[
            pl.BlockSpec((2, _BB, D), lambda i: (0, i, 0)),
            full((2 * D, H1)), full((1, H1)),
            full((H1, H2)), full((1, H2)),
            full((H2, 1)), full((1, 1)),
            full((2 * D, H1)), full((1, H1)),
            full((H1, H2)), full((1, H2)),
            full((H2, 1)), full((1, 1)),
        ],
        out_specs=pl.BlockSpec((2, _BB), lambda i: (0, i)),
        out_shape=jax.ShapeDtypeStruct((2, B), jnp.float32),
    )(pooled, cW1, cb1, cW2, cb2, cW3, cb3, vW1, vb1, vW2, vb2, vW3, vb3)


def kernel(user_idx, item_idx, user_tables, item_tables,
           ctr_W1, ctr_b1, ctr_W2, ctr_b2, ctr_W3, ctr_b3,
           cvr_W1, cvr_b1, cvr_W2, cvr_b2, cvr_W3, cvr_b3):
    ut = user_tables.reshape(NU * V, D)
    it = item_tables.reshape(NI * V, D)
    uoff = (jnp.arange(NU, dtype=jnp.int32) * V)[None, :]
    ioff = (jnp.arange(NI, dtype=jnp.int32) * V)[None, :]
    uidx = (user_idx.astype(jnp.int32) + uoff).reshape(_NW, _JPW, 128)
    iidx = (item_idx.astype(jnp.int32) + ioff).reshape(_NW, _JPW, 128)
    pooled = _pool(ut, it, uidx, iidx)
    return _mlp(pooled,
                ctr_W1, ctr_b1.reshape(1, H1), ctr_W2, ctr_b2.reshape(1, H2),
                ctr_W3, ctr_b3.reshape(1, 1),
                cvr_W1, cvr_b1.reshape(1, H1), cvr_W2, cvr_b2.reshape(1, H2),
                cvr_W3, cvr_b3.reshape(1, 1))


# speedup vs baseline: 4.4997x; 4.4997x over previous
import jax
import jax.numpy as jnp
from jax import lax
from jax.experimental import pallas as pl
from jax.experimental.pallas import tpu as pltpu
from jax.experimental.pallas import tpu_sc as plsc

B = 16384
V = 100000
D = 16
NU = 13
NI = 13
H1, H2 = 256, 128

_NC, _NS = 2, 16
_NW = _NC * _NS
_IC = 4096
_NIC = B // _IC
_BB = 4096


def _pool_body(ut, it, uidxT, iidxT, out, plane_v, idx_v, acc_v):
    w = lax.axis_index("c") * _NS + lax.axis_index("s")
    d = w % _NS

    def side_prog(tbl, idxT, nf):
        for f in range(nf):
            pltpu.sync_copy(tbl.at[f, d, :], plane_v)
            for c in range(_NIC):
                pltpu.sync_copy(idxT.at[pl.ds(f, 1), pl.ds(c * _IC, _IC)],
                                idx_v)

                def step(i, carry, f=f, c=c):
                    vidx = idx_v[0, pl.ds(i * 16, 16)]
                    g = plsc.load_gather(plane_v, [vidx])
                    if f == 0:
                        acc_v[pl.ds(c * _IC + i * 16, 16)] = g
                    else:
                        plsc.addupdate(acc_v.at[pl.ds(c * _IC + i * 16, 16)], g)
                    return carry

                lax.fori_loop(0, _IC // 16, step, 0, unroll=8)

    @pl.when(w < _NS)
    def _():
        side_prog(ut, uidxT, NU)

    @pl.when(w >= _NS)
    def _():
        side_prog(it, iidxT, NI)

    pltpu.sync_copy(acc_v, out.at[w])


def _pool(ut, it, uidxT, iidxT):
    mesh = plsc.VectorSubcoreMesh(core_axis_name="c", subcore_axis_name="s")
    f = pl.kernel(
        _pool_body,
        out_type=jax.ShapeDtypeStruct((_NW, B), jnp.float32),
        mesh=mesh,
        scratch_types=[
            pltpu.VMEM((V,), jnp.float32),
            pltpu.VMEM((1, _IC), jnp.int32),
            pltpu.VMEM((B,), jnp.float32),
        ],
        compiler_params=pltpu.CompilerParams(
            use_tc_tiling_on_sc=True, needs_layout_passes=False),
    )
    return f(ut, it, uidxT, iidxT)


def _mlp_body(x, cW1, cb1, cW2, cb2, cW3, cb3, vW1, vb1, vW2, vb2, vW3, vb3,
              out):
    xu = x[0:D, :]
    xi = x[D:2 * D, :]
    cdim = (((0,), (0,)), ((), ()))

    def tower(W1, b1, W2, b2, W3, b3):
        h = lax.dot_general(xu, W1[0:D, :], cdim,
                            preferred_element_type=jnp.float32)
        h = h + lax.dot_general(xi, W1[D:2 * D, :], cdim,
                                preferred_element_type=jnp.float32)
        h = jnp.maximum(h + b1[0, :], 0.0)
        h = jnp.dot(h, W2[...], preferred_element_type=jnp.float32) + b2[0, :]
        h = jnp.maximum(h, 0.0)
        o = jnp.dot(h, W3[...], preferred_element_type=jnp.float32)
        return o[:, 0] + b3[0, 0]

    ctr = jax.nn.sigmoid(tower(cW1, cb1, cW2, cb2, cW3, cb3))
    cvr = jax.nn.sigmoid(tower(vW1, vb1, vW2, vb2, vW3, vb3))
    out[0, :] = ctr
    out[1, :] = ctr * cvr


def _mlp(pooled, cW1, cb1, cW2, cb2, cW3, cb3, vW1, vb1, vW2, vb2, vW3, vb3):
    full = lambda shape: pl.BlockSpec(shape, lambda i: (0,) * len(shape))
    return pl.pallas_call(
        _mlp_body,
        grid=(B // _BB,),
        in_specs=[
            pl.BlockSpec((_NW, _BB), lambda i: (0, i)),
            full((2 * D, H1)), full((1, H1)),
            full((H1, H2)), full((1, H2)),
            full((H2, 1)), full((1, 1)),
            full((2 * D, H1)), full((1, H1)),
            full((H1, H2)), full((1, H2)),
            full((H2, 1)), full((1, 1)),
        ],
        out_specs=pl.BlockSpec((2, _BB), lambda i: (0, i)),
        out_shape=jax.ShapeDtypeStruct((2, B), jnp.float32),
    )(pooled, cW1, cb1, cW2, cb2, cW3, cb3, vW1, vb1, vW2, vb2, vW3, vb3)


def kernel(user_idx, item_idx, user_tables, item_tables,
           ctr_W1, ctr_b1, ctr_W2, ctr_b2, ctr_W3, ctr_b3,
           cvr_W1, cvr_b1, cvr_W2, cvr_b2, cvr_W3, cvr_b3):
    ut = jnp.swapaxes(user_tables, 1, 2)
    it = jnp.swapaxes(item_tables, 1, 2)
    uidxT = user_idx.astype(jnp.int32).T
    iidxT = item_idx.astype(jnp.int32).T
    pooled = _pool(ut, it, uidxT, iidxT)
    return _mlp(pooled,
                ctr_W1, ctr_b1.reshape(1, H1), ctr_W2, ctr_b2.reshape(1, H2),
                ctr_W3, ctr_b3.reshape(1, 1),
                cvr_W1, cvr_b1.reshape(1, H1), cvr_W2, cvr_b2.reshape(1, H2),
                cvr_W3, cvr_b3.reshape(1, 1))


# speedup vs baseline: 5.3675x; 1.1929x over previous
import jax
import jax.numpy as jnp
from jax import lax
from jax.experimental import pallas as pl
from jax.experimental.pallas import tpu as pltpu
from jax.experimental.pallas import tpu_sc as plsc

B = 16384
V = 100000
D = 16
NU = 13
NI = 13
H1, H2 = 256, 128

_NC, _NS = 2, 16
_NW = _NC * _NS
_IC = 4096
_NIC = B // _IC
_BB = 4096


def _pool_body(ut, it, uidxT, iidxT, out, plane_v, idx_v, acc_v):
    w = lax.axis_index("c") * _NS + lax.axis_index("s")
    d = w % _NS

    def side_prog(tbl, idxT, nf):
        for f in range(nf):
            pltpu.sync_copy(tbl.at[f, d, :], plane_v)
            for c in range(_NIC):
                pltpu.sync_copy(idxT.at[pl.ds(f, 1), pl.ds(c * _IC, _IC)],
                                idx_v)

                def make_step(f, c):
                    def step(i):
                        vidx = idx_v[0, pl.ds(i * 16, 16)]
                        g = plsc.load_gather(plane_v, [vidx])
                        if f == 0:
                            acc_v[pl.ds(c * _IC + i * 16, 16)] = g
                        else:
                            plsc.addupdate(
                                acc_v.at[pl.ds(c * _IC + i * 16, 16)], g)
                    return step

                plsc.parallel_loop(0, _IC // 16, unroll=8)(make_step(f, c))

    @pl.when(w < _NS)
    def _():
        side_prog(ut, uidxT, NU)

    @pl.when(w >= _NS)
    def _():
        side_prog(it, iidxT, NI)

    pltpu.sync_copy(acc_v, out.at[w])


def _pool(ut, it, uidxT, iidxT):
    mesh = plsc.VectorSubcoreMesh(core_axis_name="c", subcore_axis_name="s")
    f = pl.kernel(
        _pool_body,
        out_type=jax.ShapeDtypeStruct((_NW, B), jnp.float32),
        mesh=mesh,
        scratch_types=[
            pltpu.VMEM((V,), jnp.float32),
            pltpu.VMEM((1, _IC), jnp.int32),
            pltpu.VMEM((B,), jnp.float32),
        ],
        compiler_params=pltpu.CompilerParams(
            use_tc_tiling_on_sc=True, needs_layout_passes=False),
    )
    return f(ut, it, uidxT, iidxT)


def _mlp_body(x, cW1, cb1, cW2, cb2, cW3, cb3, vW1, vb1, vW2, vb2, vW3, vb3,
              out):
    xu = x[0:D, :]
    xi = x[D:2 * D, :]
    cdim = (((0,), (0,)), ((), ()))

    def tower(W1, b1, W2, b2, W3, b3):
        h = lax.dot_general(xu, W1[0:D, :], cdim,
                            preferred_element_type=jnp.float32)
        h = h + lax.dot_general(xi, W1[D:2 * D, :], cdim,
                                preferred_element_type=jnp.float32)
        h = jnp.maximum(h + b1[0, :], 0.0)
        h = jnp.dot(h, W2[...], preferred_element_type=jnp.float32) + b2[0, :]
        h = jnp.maximum(h, 0.0)
        o = jnp.dot(h, W3[...], preferred_element_type=jnp.float32)
        return o[:, 0] + b3[0, 0]

    ctr = jax.nn.sigmoid(tower(cW1, cb1, cW2, cb2, cW3, cb3))
    cvr = jax.nn.sigmoid(tower(vW1, vb1, vW2, vb2, vW3, vb3))
    out[0, :] = ctr
    out[1, :] = ctr * cvr


def _mlp(pooled, cW1, cb1, cW2, cb2, cW3, cb3, vW1, vb1, vW2, vb2, vW3, vb3):
    full = lambda shape: pl.BlockSpec(shape, lambda i: (0,) * len(shape))
    return pl.pallas_call(
        _mlp_body,
        grid=(B // _BB,),
        in_specs=[
            pl.BlockSpec((_NW, _BB), lambda i: (0, i)),
            full((2 * D, H1)), full((1, H1)),
            full((H1, H2)), full((1, H2)),
            full((H2, 1)), full((1, 1)),
            full((2 * D, H1)), full((1, H1)),
            full((H1, H2)), full((1, H2)),
            full((H2, 1)), full((1, 1)),
        ],
        out_specs=pl.BlockSpec((2, _BB), lambda i: (0, i)),
        out_shape=jax.ShapeDtypeStruct((2, B), jnp.float32),
    )(pooled, cW1, cb1, cW2, cb2, cW3, cb3, vW1, vb1, vW2, vb2, vW3, vb3)


def kernel(user_idx, item_idx, user_tables, item_tables,
           ctr_W1, ctr_b1, ctr_W2, ctr_b2, ctr_W3, ctr_b3,
           cvr_W1, cvr_b1, cvr_W2, cvr_b2, cvr_W3, cvr_b3):
    ut = jnp.swapaxes(user_tables, 1, 2)
    it = jnp.swapaxes(item_tables, 1, 2)
    uidxT = user_idx.astype(jnp.int32).T
    iidxT = item_idx.astype(jnp.int32).T
    pooled = _pool(ut, it, uidxT, iidxT)
    return _mlp(pooled,
                ctr_W1, ctr_b1.reshape(1, H1), ctr_W2, ctr_b2.reshape(1, H2),
                ctr_W3, ctr_b3.reshape(1, 1),
                cvr_W1, cvr_b1.reshape(1, H1), cvr_W2, cvr_b2.reshape(1, H2),
                cvr_W3, cvr_b3.reshape(1, 1))


# speedup vs baseline: 6.1566x; 1.1470x over previous
import jax
import jax.numpy as jnp
from jax import lax
from jax.experimental import pallas as pl
from jax.experimental.pallas import tpu as pltpu
from jax.experimental.pallas import tpu_sc as plsc

B = 16384
V = 100000
D = 16
NU = 13
NI = 13
H1, H2 = 256, 128

_NC, _NS = 2, 16
_NW = _NC * _NS
_IC = 4096
_NIC = B // _IC
_BB = 4096


def _pool_body(ut, it, uidxT, iidxT, out, plane_v, idx_v, acc_v, psem, isem):
    w = lax.axis_index("c") * _NS + lax.axis_index("s")
    d = w % _NS

    def side_prog(tbl, idxT, nf):
        def start_plane(f):
            return [pltpu.async_copy(tbl.at[f, d, :], plane_v, psem)]

        def start_idx(f, c):
            return pltpu.async_copy(
                idxT.at[pl.ds(f, 1), pl.ds(c * _IC, _IC)],
                idx_v.at[(f * _NIC + c) % 2], isem)

        def make_step(f, c):
            def step(i):
                vidx = idx_v[(f * _NIC + c) % 2, 0, pl.ds(i * 16, 16)]
                g = plsc.load_gather(plane_v, [vidx])
                if f == 0:
                    acc_v[pl.ds(c * _IC + i * 16, 16)] = g
                else:
                    plsc.addupdate(acc_v.at[pl.ds(c * _IC + i * 16, 16)], g)
            return step

        plane_cps = start_plane(0)
        idx_cp = start_idx(0, 0)
        for f in range(nf):
            for cp in plane_cps:
                cp.wait()
            for c in range(_NIC):
                idx_cp.wait()
                if c + 1 < _NIC:
                    idx_cp = start_idx(f, c + 1)
                elif f + 1 < nf:
                    idx_cp = start_idx(f + 1, 0)
                plsc.parallel_loop(0, _IC // 16, unroll=8)(make_step(f, c))
            if f + 1 < nf:
                plane_cps = start_plane(f + 1)

    @pl.when(w < _NS)
    def _():
        side_prog(ut, uidxT, NU)

    @pl.when(w >= _NS)
    def _():
        side_prog(it, iidxT, NI)

    pltpu.sync_copy(acc_v, out.at[w])


def _pool(ut, it, uidxT, iidxT):
    mesh = plsc.VectorSubcoreMesh(core_axis_name="c", subcore_axis_name="s")
    f = pl.kernel(
        _pool_body,
        out_type=jax.ShapeDtypeStruct((_NW, B), jnp.float32),
        mesh=mesh,
        scratch_types=[
            pltpu.VMEM((V,), jnp.float32),
            pltpu.VMEM((2, 1, _IC), jnp.int32),
            pltpu.VMEM((B,), jnp.float32),
            pltpu.SemaphoreType.DMA,
            pltpu.SemaphoreType.DMA,
        ],
        compiler_params=pltpu.CompilerParams(
            use_tc_tiling_on_sc=True, needs_layout_passes=False),
    )
    return f(ut, it, uidxT, iidxT)


def _mlp_body(x, cW1, cb1, cW2, cb2, cW3, cb3, vW1, vb1, vW2, vb2, vW3, vb3,
              out):
    xu = x[0:D, :]
    xi = x[D:2 * D, :]
    cdim = (((0,), (0,)), ((), ()))

    def tower(W1, b1, W2, b2, W3, b3):
        h = lax.dot_general(xu, W1[0:D, :], cdim,
                            preferred_element_type=jnp.float32)
        h = h + lax.dot_general(xi, W1[D:2 * D, :], cdim,
                                preferred_element_type=jnp.float32)
        h = jnp.maximum(h + b1[0, :], 0.0)
        h = jnp.dot(h, W2[...], preferred_element_type=jnp.float32) + b2[0, :]
        h = jnp.maximum(h, 0.0)
        o = jnp.dot(h, W3[...], preferred_element_type=jnp.float32)
        return o[:, 0] + b3[0, 0]

    ctr = jax.nn.sigmoid(tower(cW1, cb1, cW2, cb2, cW3, cb3))
    cvr = jax.nn.sigmoid(tower(vW1, vb1, vW2, vb2, vW3, vb3))
    out[0, :] = ctr
    out[1, :] = ctr * cvr


def _mlp(pooled, cW1, cb1, cW2, cb2, cW3, cb3, vW1, vb1, vW2, vb2, vW3, vb3):
    full = lambda shape: pl.BlockSpec(shape, lambda i: (0,) * len(shape))
    return pl.pallas_call(
        _mlp_body,
        grid=(B // _BB,),
        in_specs=[
            pl.BlockSpec((_NW, _BB), lambda i: (0, i)),
            full((2 * D, H1)), full((1, H1)),
            full((H1, H2)), full((1, H2)),
            full((H2, 1)), full((1, 1)),
            full((2 * D, H1)), full((1, H1)),
            full((H1, H2)), full((1, H2)),
            full((H2, 1)), full((1, 1)),
        ],
        out_specs=pl.BlockSpec((2, _BB), lambda i: (0, i)),
        out_shape=jax.ShapeDtypeStruct((2, B), jnp.float32),
    )(pooled, cW1, cb1, cW2, cb2, cW3, cb3, vW1, vb1, vW2, vb2, vW3, vb3)


def kernel(user_idx, item_idx, user_tables, item_tables,
           ctr_W1, ctr_b1, ctr_W2, ctr_b2, ctr_W3, ctr_b3,
           cvr_W1, cvr_b1, cvr_W2, cvr_b2, cvr_W3, cvr_b3):
    ut = jnp.swapaxes(user_tables, 1, 2)
    it = jnp.swapaxes(item_tables, 1, 2)
    uidxT = user_idx.astype(jnp.int32).T
    iidxT = item_idx.astype(jnp.int32).T
    pooled = _pool(ut, it, uidxT, iidxT)
    return _mlp(pooled,
                ctr_W1, ctr_b1.reshape(1, H1), ctr_W2, ctr_b2.reshape(1, H2),
                ctr_W3, ctr_b3.reshape(1, 1),
                cvr_W1, cvr_b1.reshape(1, H1), cvr_W2, cvr_b2.reshape(1, H2),
                cvr_W3, cvr_b3.reshape(1, 1))


# speedup vs baseline: 6.6418x; 1.0788x over previous
import jax
import jax.numpy as jnp
from jax import lax
from jax.experimental import pallas as pl
from jax.experimental.pallas import tpu as pltpu
from jax.experimental.pallas import tpu_sc as plsc

B = 16384
V = 100000
D = 16
NU = 13
NI = 13
H1, H2 = 256, 128

_NC, _NS = 2, 16
_NW = _NC * _NS
_IC = 4096
_NIC = B // _IC
_BB = 8192


def _pool_body(ut, it, uidxT, iidxT, out, plane_v, idx_v, acc_v, psem, isem):
    w = lax.axis_index("c") * _NS + lax.axis_index("s")
    d = w % _NS

    def side_prog(tbl, idxT, nf):
        def start_plane(f):
            return [pltpu.async_copy(tbl.at[f, d, :], plane_v, psem)]

        def start_idx(f, c):
            return pltpu.async_copy(
                idxT.at[pl.ds(f, 1), pl.ds(c * _IC, _IC)],
                idx_v.at[(f * _NIC + c) % 2], isem)

        def make_step(f, c):
            def step(i):
                vidx = idx_v[(f * _NIC + c) % 2, 0, pl.ds(i * 16, 16)]
                g = plsc.load_gather(plane_v, [vidx])
                if f == 0:
                    acc_v[pl.ds(c * _IC + i * 16, 16)] = g
                else:
                    plsc.addupdate(acc_v.at[pl.ds(c * _IC + i * 16, 16)], g)
            return step

        plane_cps = start_plane(0)
        idx_cp = start_idx(0, 0)
        for f in range(nf):
            for cp in plane_cps:
                cp.wait()
            for c in range(_NIC):
                idx_cp.wait()
                if c + 1 < _NIC:
                    idx_cp = start_idx(f, c + 1)
                elif f + 1 < nf:
                    idx_cp = start_idx(f + 1, 0)
                plsc.parallel_loop(0, _IC // 16, unroll=8)(make_step(f, c))
            if f + 1 < nf:
                plane_cps = start_plane(f + 1)

    @pl.when(w < _NS)
    def _():
        side_prog(ut, uidxT, NU)

    @pl.when(w >= _NS)
    def _():
        side_prog(it, iidxT, NI)

    pltpu.sync_copy(acc_v, out.at[w])


def _pool(ut, it, uidxT, iidxT):
    mesh = plsc.VectorSubcoreMesh(core_axis_name="c", subcore_axis_name="s")
    f = pl.kernel(
        _pool_body,
        out_type=jax.ShapeDtypeStruct((_NW, B), jnp.float32),
        mesh=mesh,
        scratch_types=[
            pltpu.VMEM((V,), jnp.float32),
            pltpu.VMEM((2, 1, _IC), jnp.int32),
            pltpu.VMEM((B,), jnp.float32),
            pltpu.SemaphoreType.DMA,
            pltpu.SemaphoreType.DMA,
        ],
        compiler_params=pltpu.CompilerParams(
            use_tc_tiling_on_sc=True, needs_layout_passes=False),
    )
    return f(ut, it, uidxT, iidxT)


def _mlp_body(x, cW1, cb1, cW2, cb2, cW3, cb3, vW1, vb1, vW2, vb2, vW3, vb3,
              out):
    xf = x[...]
    cdim = (((0,), (0,)), ((), ()))

    def tower(W1, b1, W2, b2, W3, b3):
        h = lax.dot_general(xf, W1[...], cdim,
                            preferred_element_type=jnp.float32)
        h = jnp.maximum(h + b1[0, :], 0.0)
        h = jnp.dot(h, W2[...], preferred_element_type=jnp.float32) + b2[0, :]
        h = jnp.maximum(h, 0.0)
        o = jnp.dot(h, W3[...], preferred_element_type=jnp.float32)
        return o[:, 0] + b3[0, 0]

    ctr = jax.nn.sigmoid(tower(cW1, cb1, cW2, cb2, cW3, cb3))
    cvr = jax.nn.sigmoid(tower(vW1, vb1, vW2, vb2, vW3, vb3))
    out[0, :] = ctr
    out[1, :] = ctr * cvr


def _mlp(pooled, cW1, cb1, cW2, cb2, cW3, cb3, vW1, vb1, vW2, vb2, vW3, vb3):
    full = lambda shape: pl.BlockSpec(shape, lambda i: (0,) * len(shape))
    return pl.pallas_call(
        _mlp_body,
        grid=(B // _BB,),
        in_specs=[
            pl.BlockSpec((_NW, _BB), lambda i: (0, i)),
            full((2 * D, H1)), full((1, H1)),
            full((H1, H2)), full((1, H2)),
            full((H2, 1)), full((1, 1)),
            full((2 * D, H1)), full((1, H1)),
            full((H1, H2)), full((1, H2)),
            full((H2, 1)), full((1, 1)),
        ],
        out_specs=pl.BlockSpec((2, _BB), lambda i: (0, i)),
        out_shape=jax.ShapeDtypeStruct((2, B), jnp.float32),
    )(pooled, cW1, cb1, cW2, cb2, cW3, cb3, vW1, vb1, vW2, vb2, vW3, vb3)


def kernel(user_idx, item_idx, user_tables, item_tables,
           ctr_W1, ctr_b1, ctr_W2, ctr_b2, ctr_W3, ctr_b3,
           cvr_W1, cvr_b1, cvr_W2, cvr_b2, cvr_W3, cvr_b3):
    ut = jnp.swapaxes(user_tables, 1, 2)
    it = jnp.swapaxes(item_tables, 1, 2)
    uidxT = user_idx.astype(jnp.int32).T
    iidxT = item_idx.astype(jnp.int32).T
    pooled = _pool(ut, it, uidxT, iidxT)
    return _mlp(pooled,
                ctr_W1, ctr_b1.reshape(1, H1), ctr_W2, ctr_b2.reshape(1, H2),
                ctr_W3, ctr_b3.reshape(1, 1),
                cvr_W1, cvr_b1.reshape(1, H1), cvr_W2, cvr_b2.reshape(1, H2),
                cvr_W3, cvr_b3.reshape(1, 1))
